# baseline (speedup 1.0000x reference)
import functools

import jax
import jax.numpy as jnp
from jax import lax
from jax.experimental import pallas as pl
from jax.experimental.pallas import tpu as pltpu
from jax.experimental.pallas import tpu_sc as plsc

N = 10000
E = 320000
D = 128
R = 200
NUM_NF = 5

NC = 2
NS = 16
DH = D // NC
EW = 100
CR = 2
CE = EW * CR
NCHUNKS = E // CE
CPT = NCHUNKS // NS
EPAD = 32
WPT = 624
TAIL = N - WPT * NS
DEGT = 10
LANES = 16


def _sc_body(x_hbm, rel_hbm, eil_hbm,
             a1_hbm, a2_hbm, deg_hbm,
             relh, a1h, a2h, degh,
             xr0, xr1, rr0, eb0, eb1, eb2, eb3,
             onesb, iotab, vbuf,
             esem, gsem, ssem, dsem):
    c = lax.axis_index("c")
    s = lax.axis_index("s")
    XR = [xr0, xr1]
    RR = [rr0, rr0]
    EB = [eb0, eb1, eb2, eb3]

    def eidx(k):
        return c * NCHUNKS + s + k * NS

    def _zero(r, carry):
        for cc in range(DH // LANES):
            xr0[r, pl.ds(cc * LANES, LANES)] = jnp.zeros((LANES,), jnp.float32)
        return carry
    lax.fori_loop(0, CE, _zero, 0)

    base = s * WPT
    for off in range(0, WPT, CE):
        seg = min(CE, WPT - off)
        pltpu.sync_copy(xr0.at[pl.ds(0, seg)], a1h.at[pl.ds(base + off, seg)])
        pltpu.sync_copy(xr0.at[pl.ds(0, seg)], a2h.at[pl.ds(base + off, seg)])

    @pl.when(s == NS - 1)
    def _():
        tb = NS * WPT
        pltpu.sync_copy(xr0.at[pl.ds(0, TAIL)], a1h.at[pl.ds(tb, TAIL)])
        pltpu.sync_copy(xr0.at[pl.ds(0, TAIL)], a2h.at[pl.ds(tb, TAIL)])

    @pl.when(s == 0)
    def _():
        pltpu.sync_copy(rel_hbm.at[c], relh)

    for k in range(7):
        onesb[pl.ds(k * LANES, LANES)] = jnp.ones((LANES,), jnp.float32)

    @pl.when(jnp.logical_and(c == 0, s < DEGT))
    def _():
        for q in range(8):
            for k in range(8):
                v = s * 1024 + q * 128 + k * LANES + lax.iota(jnp.int32, LANES)
                iotab[q, pl.ds(k * LANES, LANES)] = jnp.minimum(v, N - 1)
                vbuf[q, pl.ds(k * LANES, LANES)] = jnp.zeros((LANES,), jnp.float32)
        dz = [pltpu.async_copy(vbuf.at[q], degh.at[iotab.at[q]], dsem)
              for q in range(8)]
        for d in dz:
            d.wait()

    plsc.subcore_barrier()

    def issue_xg(P, ei):
        for j in range(CR):
            pltpu.async_copy(x_hbm.at[EB[ei].at[j]],
                             XR[P].at[pl.ds(j * EW, EW)], gsem)

    def wait_xg(P, ei):
        for j in range(CR):
            pltpu.make_async_copy(x_hbm.at[EB[ei].at[j]],
                                  XR[P].at[pl.ds(j * EW, EW)], gsem).wait()

    def issue_sc(P, ei):
        for j in range(CR):
            pltpu.async_copy(XR[P].at[pl.ds(j * EW, EW)],
                             a1h.at[EB[ei].at[CR + j]], ssem, add=True)
            pltpu.async_copy(RR[P].at[pl.ds(j * EW, EW)],
                             a2h.at[EB[ei].at[CR + j]], ssem, add=True)

    def wait_sc(P, ei):
        for j in range(CR):
            pltpu.make_async_copy(XR[P].at[pl.ds(j * EW, EW)],
                                  a1h.at[EB[ei].at[CR + j]], ssem).wait()
            pltpu.make_async_copy(RR[P].at[pl.ds(j * EW, EW)],
                                  a2h.at[EB[ei].at[CR + j]], ssem).wait()

    def body(k, P, ei):
        Q = 1 - P
        wait_xg(P, ei)

        @pl.when(k > 0)
        def _():
            wait_sc(Q, (ei + 3) % 4)

        pltpu.make_async_copy(eil_hbm.at[eidx(k + 1)], EB[(ei + 1) % 4],
                              esem).wait()

        rg = [pltpu.async_copy(relh.at[EB[ei].at[2 * CR + j]],
                               RR[P].at[pl.ds(j * EW, EW)], gsem)
              for j in range(CR)]
        for d in rg:
            d.wait()

        @pl.when(k < CPT - 1)
        def _():
            issue_xg(Q, (ei + 1) % 4)

        pltpu.async_copy(eil_hbm.at[eidx(k + 2)], EB[(ei + 2) % 4], esem)

        @pl.when(c == 0)
        def _():
            for j in range(CR):
                pltpu.async_copy(onesb.at[pl.ds(0, EW)],
                                 degh.at[EB[ei].at[CR + j]], dsem, add=True)

        xr, rr = XR[P], RR[P]

        def _comp(r, carry2):
            for cc in range(DH // LANES):
                sl = pl.ds(cc * LANES, LANES)
                xi = xr[r, sl]
                ri = rr[r, sl]
                xr[r, sl] = xi * ri
                rr[r, sl] = xi - ri
            return carry2
        lax.fori_loop(0, CE, _comp, 0, unroll=4)

        @pl.when(c == 0)
        def _():
            for j in range(CR):
                pltpu.make_async_copy(onesb.at[pl.ds(0, EW)],
                                      degh.at[EB[ei].at[CR + j]], dsem).wait()

        issue_sc(P, ei)

    pltpu.sync_copy(eil_hbm.at[eidx(0)], EB[0])
    issue_xg(0, 0)
    pltpu.async_copy(eil_hbm.at[eidx(1)], EB[1], esem)

    def outer(m, carry):
        for i in range(4):
            body(m * 4 + i, i % 2, i)
        return carry
    lax.fori_loop(0, CPT // 4, outer, 0)
    wait_sc((CPT - 1) % 2, (CPT - 1) % 4)
    pltpu.make_async_copy(eil_hbm.at[eidx(CPT + 1)],
                          EB[(CPT + 1) % 4], esem).wait()

    plsc.subcore_barrier()

    pltpu.sync_copy(a1h.at[pl.ds(base, WPT)], a1_hbm.at[c, pl.ds(base, WPT)])
    pltpu.sync_copy(a2h.at[pl.ds(base, WPT)], a2_hbm.at[c, pl.ds(base, WPT)])

    @pl.when(s == NS - 1)
    def _():
        tb = NS * WPT
        pltpu.sync_copy(a1h.at[pl.ds(tb, TAIL)], a1_hbm.at[c, pl.ds(tb, TAIL)])
        pltpu.sync_copy(a2h.at[pl.ds(tb, TAIL)], a2_hbm.at[c, pl.ds(tb, TAIL)])

    @pl.when(jnp.logical_and(c == 0, s < DEGT))
    def _():
        dg = [pltpu.async_copy(degh.at[iotab.at[q]], vbuf.at[q], dsem)
              for q in range(8)]
        for d in dg:
            d.wait()
        pltpu.sync_copy(vbuf, deg_hbm.at[pl.ds(s * 8, 8)])


_sc_edge_pass = functools.partial(
    pl.kernel,
    out_type=(
        jax.ShapeDtypeStruct((NC, N, DH), jnp.float32),
        jax.ShapeDtypeStruct((NC, N, DH), jnp.float32),
        jax.ShapeDtypeStruct((DEGT * 8, 128), jnp.float32),
    ),
    mesh=plsc.VectorSubcoreMesh(core_axis_name="c", subcore_axis_name="s"),
    compiler_params=pltpu.CompilerParams(use_tc_tiling_on_sc=False),
    scratch_types=[
        pltpu.VMEM_SHARED((R, DH), jnp.float32),
        pltpu.VMEM_SHARED((N, DH), jnp.float32),
        pltpu.VMEM_SHARED((N, DH), jnp.float32),
        pltpu.VMEM_SHARED((N,), jnp.float32),
        pltpu.VMEM((CE, DH), jnp.float32),
        pltpu.VMEM((CE, DH), jnp.float32),
        pltpu.VMEM((CE, DH), jnp.float32),
        pltpu.VMEM((3 * CR, EW), jnp.int32),
        pltpu.VMEM((3 * CR, EW), jnp.int32),
        pltpu.VMEM((3 * CR, EW), jnp.int32),
        pltpu.VMEM((3 * CR, EW), jnp.int32),
        pltpu.VMEM((7 * LANES,), jnp.float32),
        pltpu.VMEM((8, 128), jnp.int32),
        pltpu.VMEM((8, 128), jnp.float32),
        pltpu.SemaphoreType.DMA,
        pltpu.SemaphoreType.DMA,
        pltpu.SemaphoreType.DMA,
        pltpu.SemaphoreType.DMA,
    ],
)(_sc_body)


BN = 1000


def _dense_body(naw, ndw, deg_r, a1_r, a2_r, x_r, nf_r,
                wop1_r, wop2_r, wself_r, wrel1_r, wrel2_r, rel_r,
                nw1_r, nb1_r, nw2_r, nb2_r, nw3_r, nb3_r,
                out_r, relout_r):
    i = pl.program_id(0)
    w = jax.nn.softmax(naw[...], axis=-1)
    nw = jax.nn.softmax(ndw[...], axis=-1)
    w0, w1, w2 = w[0, 0], w[0, 1], w[0, 2]

    inv = 1.0 / jnp.maximum(deg_r[...], 1.0)
    a1 = jnp.concatenate([a1_r[0], a1_r[1]], axis=-1) * inv
    a2 = jnp.concatenate([a2_r[0], a2_r[1]], axis=-1) * inv
    acc = w0 * jnp.dot(a1, wop1_r[...], preferred_element_type=jnp.float32)
    acc += w1 * jnp.dot(a2, wop2_r[...], preferred_element_type=jnp.float32)
    acc += w2 * jnp.dot(x_r[...], wself_r[...],
                        preferred_element_type=jnp.float32)

    nf = nf_r[...]
    for j in range(NUM_NF):
        f = nf[:, j:j + 1]
        h = jnp.maximum(f * nw1_r[j] + nb1_r[j], 0.0)
        h = jnp.maximum(jnp.dot(h, nw2_r[j], preferred_element_type=jnp.float32)
                        + nb2_r[j], 0.0)
        h = jnp.dot(h, nw3_r[j], preferred_element_type=jnp.float32) + nb3_r[j]
        acc += nw[0, j] * h
    out_r[...] = jnp.maximum(acc, 0.0)

    @pl.when(i == 0)
    def _():
        rel = rel_r[...]
        rm = w0 * jnp.dot(rel, wrel1_r[...], preferred_element_type=jnp.float32)
        rm += w1 * jnp.dot(rel, wrel2_r[...], preferred_element_type=jnp.float32)
        rm += w2 * rel
        relout_r[...] = rm


def _dense(na_weights, node_weights, deg, a1, a2, x, node_feat,
           Wop1, Wop2, Wself, Wrel1, Wrel2, rel_embeds,
           nodeW1, nodeB1, nodeW2, nodeB2, nodeW3, nodeB3):
    cst2 = lambda i: (0, 0)
    cst3 = lambda i: (0, 0, 0)
    row2 = lambda i: (i, 0)
    half = lambda i: (0, i, 0)
    return pl.pallas_call(
        _dense_body,
        grid=(N // BN,),
        in_specs=[
            pl.BlockSpec((1, 3), cst2),
            pl.BlockSpec((1, NUM_NF), cst2),
            pl.BlockSpec((BN, 1), row2),
            pl.BlockSpec((NC, BN, DH), half),
            pl.BlockSpec((NC, BN, DH), half),
            pl.BlockSpec((BN, D), row2),
            pl.BlockSpec((BN, NUM_NF), row2),
            pl.BlockSpec((D, D), cst2),
            pl.BlockSpec((D, D), cst2),
            pl.BlockSpec((D, D), cst2),
            pl.BlockSpec((D, D), cst2),
            pl.BlockSpec((D, D), cst2),
            pl.BlockSpec((R, D), cst2),
            pl.BlockSpec((NUM_NF, D), cst2),
            pl.BlockSpec((NUM_NF, D), cst2),
            pl.BlockSpec((NUM_NF, D, D), cst3),
            pl.BlockSpec((NUM_NF, D), cst2),
            pl.BlockSpec((NUM_NF, D, D), cst3),
            pl.BlockSpec((NUM_NF, D), cst2),
        ],
        out_specs=[
            pl.BlockSpec((BN, D), row2),
            pl.BlockSpec((R, D), cst2),
        ],
        out_shape=(
            jax.ShapeDtypeStruct((N, D), jnp.float32),
            jax.ShapeDtypeStruct((R, D), jnp.float32),
        ),
    )(na_weights.reshape(1, 3), node_weights.reshape(1, NUM_NF),
      deg, a1, a2, x, node_feat,
      Wop1, Wop2, Wself, Wrel1, Wrel2, rel_embeds,
      nodeW1.reshape(NUM_NF, D), nodeB1, nodeW2, nodeB2, nodeW3, nodeB3)


def kernel(x, edge_index, edge_type, rel_embeds, na_weights, node_weights,
           node_feat, Wop1, Wop2, Wself, Wrel1, Wrel2,
           nodeW1, nodeB1, nodeW2, nodeB2, nodeW3, nodeB3):
    src3 = edge_index[0].reshape(NCHUNKS, CR, EW)
    dst3 = edge_index[1].reshape(NCHUNKS, CR, EW)
    typ3 = edge_type.reshape(NCHUNKS, CR, EW)
    eil0 = jnp.stack([src3, dst3, typ3], axis=1)
    eil1 = jnp.stack([src3 + N, dst3, typ3], axis=1)
    eil = jnp.concatenate(
        [eil0, eil1, jnp.zeros((EPAD, 3, CR, EW), jnp.int32)])
    eil = eil.reshape(NC * NCHUNKS + EPAD, 3 * CR, EW)
    x2 = x.reshape(N, NC, DH).transpose(1, 0, 2).reshape(NC * N, DH)
    rel2 = rel_embeds.reshape(R, NC, DH).transpose(1, 0, 2)

    a1, a2, dego = _sc_edge_pass(x2, rel2, eil)
    deg = dego.reshape(DEGT * 1024)[:N].reshape(N, 1)

    xout, relmix = _dense(na_weights, node_weights, deg, a1, a2, x,
                          node_feat, Wop1, Wop2, Wself, Wrel1, Wrel2,
                          rel_embeds, nodeW1, nodeB1, nodeW2, nodeB2,
                          nodeW3, nodeB3)
    return xout, relmix


# speedup vs baseline: 1.0099x; 1.0099x over previous
import functools

import jax
import jax.numpy as jnp
from jax import lax
from jax.experimental import pallas as pl
from jax.experimental.pallas import tpu as pltpu
from jax.experimental.pallas import tpu_sc as plsc

N = 10000
E = 320000
D = 128
R = 200
NUM_NF = 5

NC = 2
NS = 16
DH = D // NC
EW = 100
CR = 2
CE = EW * CR
NCHUNKS = E // CE
CPT = NCHUNKS // NS
EPAD = 32
WPT = 624
TAIL = N - WPT * NS
DEGT = 10
LANES = 16


def _sc_body(x_hbm, rel_hbm, eil_hbm,
             a1_hbm, a2_hbm, deg_hbm,
             relh, a1h, a2h, degh,
             xr0, xr1, rr0, eb0, eb1, eb2, eb3,
             onesb, iotab, vbuf,
             esem, gsem, ssem, dsem):
    c = lax.axis_index("c")
    s = lax.axis_index("s")
    XR = [xr0, xr1]
    RR = [rr0, rr0]
    EB = [eb0, eb1, eb2, eb3]

    def eidx(k):
        return c * NCHUNKS + s + k * NS

    def _zero(r, carry):
        for cc in range(DH // LANES):
            xr0[r, pl.ds(cc * LANES, LANES)] = jnp.zeros((LANES,), jnp.float32)
        return carry
    lax.fori_loop(0, CE, _zero, 0)

    base = s * WPT
    for off in range(0, WPT, CE):
        seg = min(CE, WPT - off)
        pltpu.sync_copy(xr0.at[pl.ds(0, seg)], a1h.at[pl.ds(base + off, seg)])
        pltpu.sync_copy(xr0.at[pl.ds(0, seg)], a2h.at[pl.ds(base + off, seg)])

    @pl.when(s == NS - 1)
    def _():
        tb = NS * WPT
        pltpu.sync_copy(xr0.at[pl.ds(0, TAIL)], a1h.at[pl.ds(tb, TAIL)])
        pltpu.sync_copy(xr0.at[pl.ds(0, TAIL)], a2h.at[pl.ds(tb, TAIL)])

    @pl.when(s == 0)
    def _():
        pltpu.sync_copy(rel_hbm.at[c], relh)

    for k in range(7):
        onesb[pl.ds(k * LANES, LANES)] = jnp.ones((LANES,), jnp.float32)

    @pl.when(jnp.logical_and(c == 0, s < DEGT))
    def _():
        for q in range(8):
            for k in range(8):
                v = s * 1024 + q * 128 + k * LANES + lax.iota(jnp.int32, LANES)
                iotab[q, pl.ds(k * LANES, LANES)] = jnp.minimum(v, N - 1)
                vbuf[q, pl.ds(k * LANES, LANES)] = jnp.zeros((LANES,), jnp.float32)
        dz = [pltpu.async_copy(vbuf.at[q], degh.at[iotab.at[q]], dsem)
              for q in range(8)]
        for d in dz:
            d.wait()

    plsc.subcore_barrier()

    def issue_xg(P, ei):
        for j in range(CR):
            pltpu.async_copy(x_hbm.at[EB[ei].at[j]],
                             XR[P].at[pl.ds(j * EW, EW)], gsem)

    def wait_xg(P, ei):
        for j in range(CR):
            pltpu.make_async_copy(x_hbm.at[EB[ei].at[j]],
                                  XR[P].at[pl.ds(j * EW, EW)], gsem).wait()

    def issue_sc(P, ei):
        for j in range(CR):
            pltpu.async_copy(XR[P].at[pl.ds(j * EW, EW)],
                             a1h.at[EB[ei].at[CR + j]], ssem, add=True)
            pltpu.async_copy(RR[P].at[pl.ds(j * EW, EW)],
                             a2h.at[EB[ei].at[CR + j]], ssem, add=True)

    def wait_sc(P, ei):
        for j in range(CR):
            pltpu.make_async_copy(XR[P].at[pl.ds(j * EW, EW)],
                                  a1h.at[EB[ei].at[CR + j]], ssem).wait()
            pltpu.make_async_copy(RR[P].at[pl.ds(j * EW, EW)],
                                  a2h.at[EB[ei].at[CR + j]], ssem).wait()

    def body(k, P, ei):
        Q = 1 - P
        wait_xg(P, ei)

        @pl.when(k > 0)
        def _():
            wait_sc(Q, (ei + 3) % 4)

        pltpu.make_async_copy(eil_hbm.at[eidx(k + 1)], EB[(ei + 1) % 4],
                              esem).wait()

        rg = [pltpu.async_copy(relh.at[EB[ei].at[2 * CR + j]],
                               RR[P].at[pl.ds(j * EW, EW)], gsem)
              for j in range(CR)]
        for d in rg:
            d.wait()

        @pl.when(k < CPT - 1)
        def _():
            issue_xg(Q, (ei + 1) % 4)

        pltpu.async_copy(eil_hbm.at[eidx(k + 2)], EB[(ei + 2) % 4], esem)

        @pl.when(c == 0)
        def _():
            for j in range(CR):
                pltpu.async_copy(onesb.at[pl.ds(0, EW)],
                                 degh.at[EB[ei].at[CR + j]], dsem, add=True)

        xr, rr = XR[P], RR[P]

        def _comp(r, carry2):
            for cc in range(DH // LANES):
                sl = pl.ds(cc * LANES, LANES)
                xi = xr[r, sl]
                ri = rr[r, sl]
                xr[r, sl] = xi * ri
                rr[r, sl] = xi - ri
            return carry2
        lax.fori_loop(0, CE, _comp, 0, unroll=4)

        @pl.when(c == 0)
        def _():
            for j in range(CR):
                pltpu.make_async_copy(onesb.at[pl.ds(0, EW)],
                                      degh.at[EB[ei].at[CR + j]], dsem).wait()

        issue_sc(P, ei)

    pltpu.sync_copy(eil_hbm.at[eidx(0)], EB[0])
    issue_xg(0, 0)
    pltpu.async_copy(eil_hbm.at[eidx(1)], EB[1], esem)

    def outer(m, carry):
        for i in range(4):
            body(m * 4 + i, i % 2, i)
        return carry
    lax.fori_loop(0, CPT // 4, outer, 0)
    wait_sc((CPT - 1) % 2, (CPT - 1) % 4)
    pltpu.make_async_copy(eil_hbm.at[eidx(CPT + 1)],
                          EB[(CPT + 1) % 4], esem).wait()

    plsc.subcore_barrier()

    pltpu.sync_copy(a1h.at[pl.ds(base, WPT)], a1_hbm.at[c, pl.ds(base, WPT)])
    pltpu.sync_copy(a2h.at[pl.ds(base, WPT)], a2_hbm.at[c, pl.ds(base, WPT)])

    @pl.when(s == NS - 1)
    def _():
        tb = NS * WPT
        pltpu.sync_copy(a1h.at[pl.ds(tb, TAIL)], a1_hbm.at[c, pl.ds(tb, TAIL)])
        pltpu.sync_copy(a2h.at[pl.ds(tb, TAIL)], a2_hbm.at[c, pl.ds(tb, TAIL)])

    @pl.when(jnp.logical_and(c == 0, s < DEGT))
    def _():
        dg = [pltpu.async_copy(degh.at[iotab.at[q]], vbuf.at[q], dsem)
              for q in range(8)]
        for d in dg:
            d.wait()
        pltpu.sync_copy(vbuf, deg_hbm.at[pl.ds(s * 8, 8)])


_sc_edge_pass = functools.partial(
    pl.kernel,
    out_type=(
        jax.ShapeDtypeStruct((NC, N, DH), jnp.float32),
        jax.ShapeDtypeStruct((NC, N, DH), jnp.float32),
        jax.ShapeDtypeStruct((DEGT * 8, 128), jnp.float32),
    ),
    mesh=plsc.VectorSubcoreMesh(core_axis_name="c", subcore_axis_name="s"),
    compiler_params=pltpu.CompilerParams(use_tc_tiling_on_sc=False),
    scratch_types=[
        pltpu.VMEM_SHARED((R, DH), jnp.float32),
        pltpu.VMEM_SHARED((N, DH), jnp.float32),
        pltpu.VMEM_SHARED((N, DH), jnp.float32),
        pltpu.VMEM_SHARED((N,), jnp.float32),
        pltpu.VMEM((CE, DH), jnp.float32),
        pltpu.VMEM((CE, DH), jnp.float32),
        pltpu.VMEM((CE, DH), jnp.float32),
        pltpu.VMEM((3 * CR, EW), jnp.int32),
        pltpu.VMEM((3 * CR, EW), jnp.int32),
        pltpu.VMEM((3 * CR, EW), jnp.int32),
        pltpu.VMEM((3 * CR, EW), jnp.int32),
        pltpu.VMEM((7 * LANES,), jnp.float32),
        pltpu.VMEM((8, 128), jnp.int32),
        pltpu.VMEM((8, 128), jnp.float32),
        pltpu.SemaphoreType.DMA,
        pltpu.SemaphoreType.DMA,
        pltpu.SemaphoreType.DMA,
        pltpu.SemaphoreType.DMA,
    ],
)(_sc_body)


BN = 1000


def _pre_body(naw, ndw, x_r, nf_r, wself_r, wrel1_r, wrel2_r, rel_r,
              nw1_r, nb1_r, nw2_r, nb2_r, nw3_r, nb3_r,
              pre_r, relout_r):
    i = pl.program_id(0)
    w = jax.nn.softmax(naw[...], axis=-1)
    nw = jax.nn.softmax(ndw[...], axis=-1)
    w2 = w[0, 2]

    acc = w2 * jnp.dot(x_r[...], wself_r[...],
                       preferred_element_type=jnp.float32)
    nf = nf_r[...]
    for j in range(NUM_NF):
        f = nf[:, j:j + 1]
        h = jnp.maximum(f * nw1_r[j] + nb1_r[j], 0.0)
        h = jnp.maximum(jnp.dot(h, nw2_r[j], preferred_element_type=jnp.float32)
                        + nb2_r[j], 0.0)
        h = jnp.dot(h, nw3_r[j], preferred_element_type=jnp.float32) + nb3_r[j]
        acc += nw[0, j] * h
    pre_r[...] = acc

    @pl.when(i == 0)
    def _():
        rel = rel_r[...]
        rm = w[0, 0] * jnp.dot(rel, wrel1_r[...],
                               preferred_element_type=jnp.float32)
        rm += w[0, 1] * jnp.dot(rel, wrel2_r[...],
                                preferred_element_type=jnp.float32)
        rm += w2 * rel
        relout_r[...] = rm


def _dense_pre(na_weights, node_weights, x, node_feat, Wself, Wrel1, Wrel2,
               rel_embeds, nodeW1, nodeB1, nodeW2, nodeB2, nodeW3, nodeB3):
    cst2 = lambda i: (0, 0)
    cst3 = lambda i: (0, 0, 0)
    row2 = lambda i: (i, 0)
    return pl.pallas_call(
        _pre_body,
        grid=(N // BN,),
        in_specs=[
            pl.BlockSpec((1, 3), cst2),
            pl.BlockSpec((1, NUM_NF), cst2),
            pl.BlockSpec((BN, D), row2),
            pl.BlockSpec((BN, NUM_NF), row2),
            pl.BlockSpec((D, D), cst2),
            pl.BlockSpec((D, D), cst2),
            pl.BlockSpec((D, D), cst2),
            pl.BlockSpec((R, D), cst2),
            pl.BlockSpec((NUM_NF, D), cst2),
            pl.BlockSpec((NUM_NF, D), cst2),
            pl.BlockSpec((NUM_NF, D, D), cst3),
            pl.BlockSpec((NUM_NF, D), cst2),
            pl.BlockSpec((NUM_NF, D, D), cst3),
            pl.BlockSpec((NUM_NF, D), cst2),
        ],
        out_specs=[
            pl.BlockSpec((BN, D), row2),
            pl.BlockSpec((R, D), cst2),
        ],
        out_shape=(
            jax.ShapeDtypeStruct((N, D), jnp.float32),
            jax.ShapeDtypeStruct((R, D), jnp.float32),
        ),
    )(na_weights.reshape(1, 3), node_weights.reshape(1, NUM_NF),
      x, node_feat, Wself, Wrel1, Wrel2, rel_embeds,
      nodeW1.reshape(NUM_NF, D), nodeB1, nodeW2, nodeB2, nodeW3, nodeB3)


def _post_body(naw, deg_r, a1_r, a2_r, pre_r, wop1_r, wop2_r, out_r):
    w = jax.nn.softmax(naw[...], axis=-1)
    inv = 1.0 / jnp.maximum(deg_r[...], 1.0)
    a1 = jnp.concatenate([a1_r[0], a1_r[1]], axis=-1) * inv
    a2 = jnp.concatenate([a2_r[0], a2_r[1]], axis=-1) * inv
    acc = pre_r[...]
    acc += w[0, 0] * jnp.dot(a1, wop1_r[...],
                             preferred_element_type=jnp.float32)
    acc += w[0, 1] * jnp.dot(a2, wop2_r[...],
                             preferred_element_type=jnp.float32)
    out_r[...] = jnp.maximum(acc, 0.0)


def _dense_post(na_weights, deg, a1, a2, pre, Wop1, Wop2):
    cst2 = lambda i: (0, 0)
    row2 = lambda i: (i, 0)
    half = lambda i: (0, i, 0)
    return pl.pallas_call(
        _post_body,
        grid=(N // BN,),
        in_specs=[
            pl.BlockSpec((1, 3), cst2),
            pl.BlockSpec((BN, 1), row2),
            pl.BlockSpec((NC, BN, DH), half),
            pl.BlockSpec((NC, BN, DH), half),
            pl.BlockSpec((BN, D), row2),
            pl.BlockSpec((D, D), cst2),
            pl.BlockSpec((D, D), cst2),
        ],
        out_specs=pl.BlockSpec((BN, D), row2),
        out_shape=jax.ShapeDtypeStruct((N, D), jnp.float32),
    )(na_weights.reshape(1, 3), deg, a1, a2, pre, Wop1, Wop2)


def kernel(x, edge_index, edge_type, rel_embeds, na_weights, node_weights,
           node_feat, Wop1, Wop2, Wself, Wrel1, Wrel2,
           nodeW1, nodeB1, nodeW2, nodeB2, nodeW3, nodeB3):
    src3 = edge_index[0].reshape(NCHUNKS, CR, EW)
    dst3 = edge_index[1].reshape(NCHUNKS, CR, EW)
    typ3 = edge_type.reshape(NCHUNKS, CR, EW)
    eil0 = jnp.stack([src3, dst3, typ3], axis=1)
    eil1 = jnp.stack([src3 + N, dst3, typ3], axis=1)
    eil = jnp.concatenate(
        [eil0, eil1, jnp.zeros((EPAD, 3, CR, EW), jnp.int32)])
    eil = eil.reshape(NC * NCHUNKS + EPAD, 3 * CR, EW)
    x2 = x.reshape(N, NC, DH).transpose(1, 0, 2).reshape(NC * N, DH)
    rel2 = rel_embeds.reshape(R, NC, DH).transpose(1, 0, 2)

    a1, a2, dego = _sc_edge_pass(x2, rel2, eil)
    pre, relmix = _dense_pre(na_weights, node_weights, x, node_feat, Wself,
                             Wrel1, Wrel2, rel_embeds, nodeW1, nodeB1,
                             nodeW2, nodeB2, nodeW3, nodeB3)
    deg = dego.reshape(DEGT * 1024)[:N].reshape(N, 1)
    xout = _dense_post(na_weights, deg, a1, a2, pre, Wop1, Wop2)
    return xout, relmix


# speedup vs baseline: 1.2280x; 1.2159x over previous
import functools

import jax
import jax.numpy as jnp
from jax import lax
from jax.experimental import pallas as pl
from jax.experimental.pallas import tpu as pltpu
from jax.experimental.pallas import tpu_sc as plsc

N = 10000
E = 320000
D = 128
R = 200
NUM_NF = 5

NC = 2
NS = 16
DH = D // NC
EW = 80
CR = 2
CE = EW * CR
NCHUNKS = E // CE
CPT = NCHUNKS // NS
EPAD = 32
WPT = 624
TAIL = N - WPT * NS
DEGT = 10
LANES = 16


def _sc_body(x_hbm, rel_hbm, eil_hbm,
             a1_hbm, a2_hbm, deg_hbm,
             a1h, a2h, degh,
             xr0, xr1, rr0, rr1, eb0, eb1, eb2, eb3,
             onesb, iotab, vbuf,
             esem, gsem, ssem, dsem):
    c = lax.axis_index("c")
    s = lax.axis_index("s")
    XR = [xr0, xr1]
    RR = [rr0, rr1]
    EB = [eb0, eb1, eb2, eb3]

    def eidx(k):
        return s + k * NS

    def _zero(r, carry):
        for cc in range(DH // LANES):
            xr0[r, pl.ds(cc * LANES, LANES)] = jnp.zeros((LANES,), jnp.float32)
        return carry
    lax.fori_loop(0, CE, _zero, 0)

    base = s * WPT
    for off in range(0, WPT, CE):
        seg = min(CE, WPT - off)
        pltpu.sync_copy(xr0.at[pl.ds(0, seg)], a1h.at[pl.ds(base + off, seg)])
        pltpu.sync_copy(xr0.at[pl.ds(0, seg)], a2h.at[pl.ds(base + off, seg)])

    @pl.when(s == NS - 1)
    def _():
        tb = NS * WPT
        pltpu.sync_copy(xr0.at[pl.ds(0, TAIL)], a1h.at[pl.ds(tb, TAIL)])
        pltpu.sync_copy(xr0.at[pl.ds(0, TAIL)], a2h.at[pl.ds(tb, TAIL)])

    for k in range(7):
        onesb[pl.ds(k * LANES, LANES)] = jnp.ones((LANES,), jnp.float32)

    @pl.when(jnp.logical_and(c == 0, s < DEGT))
    def _():
        for q in range(8):
            for k in range(8):
                v = s * 1024 + q * 128 + k * LANES + lax.iota(jnp.int32, LANES)
                iotab[q, pl.ds(k * LANES, LANES)] = jnp.minimum(v, N - 1)
                vbuf[q, pl.ds(k * LANES, LANES)] = jnp.zeros((LANES,), jnp.float32)
        dz = [pltpu.async_copy(vbuf.at[q], degh.at[iotab.at[q]], dsem)
              for q in range(8)]
        for d in dz:
            d.wait()

    plsc.subcore_barrier()

    def issue_xg(P, ei):
        for j in range(CR):
            pltpu.async_copy(x_hbm.at[c].at[EB[ei].at[j]],
                             XR[P].at[pl.ds(j * EW, EW)], gsem)
            pltpu.async_copy(rel_hbm.at[c].at[EB[ei].at[2 * CR + j]],
                             RR[P].at[pl.ds(j * EW, EW)], gsem)

    def wait_xg(P, ei):
        for j in range(CR):
            pltpu.make_async_copy(x_hbm.at[c].at[EB[ei].at[j]],
                                  XR[P].at[pl.ds(j * EW, EW)], gsem).wait()
            pltpu.make_async_copy(rel_hbm.at[c].at[EB[ei].at[2 * CR + j]],
                                  RR[P].at[pl.ds(j * EW, EW)], gsem).wait()

    def issue_sc(P, ei):
        for j in range(CR):
            pltpu.async_copy(XR[P].at[pl.ds(j * EW, EW)],
                             a1h.at[EB[ei].at[CR + j]], ssem, add=True)
            pltpu.async_copy(RR[P].at[pl.ds(j * EW, EW)],
                             a2h.at[EB[ei].at[CR + j]], ssem, add=True)

    def wait_sc(P, ei):
        for j in range(CR):
            pltpu.make_async_copy(XR[P].at[pl.ds(j * EW, EW)],
                                  a1h.at[EB[ei].at[CR + j]], ssem).wait()
            pltpu.make_async_copy(RR[P].at[pl.ds(j * EW, EW)],
                                  a2h.at[EB[ei].at[CR + j]], ssem).wait()

    def body(k, P, ei):
        Q = 1 - P
        wait_xg(P, ei)

        @pl.when(k > 0)
        def _():
            wait_sc(Q, (ei + 3) % 4)

        pltpu.make_async_copy(eil_hbm.at[eidx(k + 1)], EB[(ei + 1) % 4],
                              esem).wait()

        @pl.when(k < CPT - 1)
        def _():
            issue_xg(Q, (ei + 1) % 4)

        pltpu.async_copy(eil_hbm.at[eidx(k + 2)], EB[(ei + 2) % 4], esem)

        @pl.when(c == 0)
        def _():
            for j in range(CR):
                pltpu.async_copy(onesb.at[pl.ds(0, EW)],
                                 degh.at[EB[ei].at[CR + j]], dsem, add=True)

        xr, rr = XR[P], RR[P]

        def _comp(r, carry2):
            for cc in range(DH // LANES):
                sl = pl.ds(cc * LANES, LANES)
                xi = xr[r, sl]
                ri = rr[r, sl]
                xr[r, sl] = xi * ri
                rr[r, sl] = xi - ri
            return carry2
        lax.fori_loop(0, CE, _comp, 0, unroll=4)

        @pl.when(c == 0)
        def _():
            for j in range(CR):
                pltpu.make_async_copy(onesb.at[pl.ds(0, EW)],
                                      degh.at[EB[ei].at[CR + j]], dsem).wait()

        issue_sc(P, ei)

    pltpu.sync_copy(eil_hbm.at[eidx(0)], EB[0])
    issue_xg(0, 0)
    pltpu.async_copy(eil_hbm.at[eidx(1)], EB[1], esem)

    def outer(m, carry):
        for i in range(4):
            body(m * 4 + i, i % 2, i)
        return carry
    lax.fori_loop(0, CPT // 4, outer, 0)
    body(CPT - 1, 0, 0)
    wait_sc(0, 0)
    pltpu.make_async_copy(eil_hbm.at[eidx(CPT + 1)],
                          EB[(CPT + 1) % 4], esem).wait()

    plsc.subcore_barrier()

    pltpu.sync_copy(a1h.at[pl.ds(base, WPT)], a1_hbm.at[c, pl.ds(base, WPT)])
    pltpu.sync_copy(a2h.at[pl.ds(base, WPT)], a2_hbm.at[c, pl.ds(base, WPT)])

    @pl.when(s == NS - 1)
    def _():
        tb = NS * WPT
        pltpu.sync_copy(a1h.at[pl.ds(tb, TAIL)], a1_hbm.at[c, pl.ds(tb, TAIL)])
        pltpu.sync_copy(a2h.at[pl.ds(tb, TAIL)], a2_hbm.at[c, pl.ds(tb, TAIL)])

    @pl.when(jnp.logical_and(c == 0, s < DEGT))
    def _():
        dg = [pltpu.async_copy(degh.at[iotab.at[q]], vbuf.at[q], dsem)
              for q in range(8)]
        for d in dg:
            d.wait()
        pltpu.sync_copy(vbuf, deg_hbm.at[pl.ds(s * 8, 8)])


_sc_edge_pass = functools.partial(
    pl.kernel,
    out_type=(
        jax.ShapeDtypeStruct((NC, N, DH), jnp.float32),
        jax.ShapeDtypeStruct((NC, N, DH), jnp.float32),
        jax.ShapeDtypeStruct((DEGT * 8, 128), jnp.float32),
    ),
    mesh=plsc.VectorSubcoreMesh(core_axis_name="c", subcore_axis_name="s"),
    compiler_params=pltpu.CompilerParams(use_tc_tiling_on_sc=False),
    scratch_types=[
        pltpu.VMEM_SHARED((N, DH), jnp.float32),
        pltpu.VMEM_SHARED((N, DH), jnp.float32),
        pltpu.VMEM_SHARED((N,), jnp.float32),
        pltpu.VMEM((CE, DH), jnp.float32),
        pltpu.VMEM((CE, DH), jnp.float32),
        pltpu.VMEM((CE, DH), jnp.float32),
        pltpu.VMEM((CE, DH), jnp.float32),
        pltpu.VMEM((3 * CR, EW), jnp.int32),
        pltpu.VMEM((3 * CR, EW), jnp.int32),
        pltpu.VMEM((3 * CR, EW), jnp.int32),
        pltpu.VMEM((3 * CR, EW), jnp.int32),
        pltpu.VMEM((7 * LANES,), jnp.float32),
        pltpu.VMEM((8, 128), jnp.int32),
        pltpu.VMEM((8, 128), jnp.float32),
        pltpu.SemaphoreType.DMA,
        pltpu.SemaphoreType.DMA,
        pltpu.SemaphoreType.DMA,
        pltpu.SemaphoreType.DMA,
    ],
)(_sc_body)


BN = 1000


def _pre_body(naw, ndw, x_r, nf_r, wself_r, wrel1_r, wrel2_r, rel_r,
              nw1_r, nb1_r, nw2_r, nb2_r, nw3_r, nb3_r,
              pre_r, relout_r):
    i = pl.program_id(0)
    w = jax.nn.softmax(naw[...], axis=-1)
    nw = jax.nn.softmax(ndw[...], axis=-1)
    w2 = w[0, 2]

    acc = w2 * jnp.dot(x_r[...], wself_r[...],
                       preferred_element_type=jnp.float32)
    nf = nf_r[...]
    for j in range(NUM_NF):
        f = nf[:, j:j + 1]
        h = jnp.maximum(f * nw1_r[j] + nb1_r[j], 0.0)
        h = jnp.maximum(jnp.dot(h, nw2_r[j], preferred_element_type=jnp.float32)
                        + nb2_r[j], 0.0)
        h = jnp.dot(h, nw3_r[j], preferred_element_type=jnp.float32) + nb3_r[j]
        acc += nw[0, j] * h
    pre_r[...] = acc

    @pl.when(i == 0)
    def _():
        rel = rel_r[...]
        rm = w[0, 0] * jnp.dot(rel, wrel1_r[...],
                               preferred_element_type=jnp.float32)
        rm += w[0, 1] * jnp.dot(rel, wrel2_r[...],
                                preferred_element_type=jnp.float32)
        rm += w2 * rel
        relout_r[...] = rm


def _dense_pre(na_weights, node_weights, x, node_feat, Wself, Wrel1, Wrel2,
               rel_embeds, nodeW1, nodeB1, nodeW2, nodeB2, nodeW3, nodeB3):
    cst2 = lambda i: (0, 0)
    cst3 = lambda i: (0, 0, 0)
    row2 = lambda i: (i, 0)
    return pl.pallas_call(
        _pre_body,
        grid=(N // BN,),
        in_specs=[
            pl.BlockSpec((1, 3), cst2),
            pl.BlockSpec((1, NUM_NF), cst2),
            pl.BlockSpec((BN, D), row2),
            pl.BlockSpec((BN, NUM_NF), row2),
            pl.BlockSpec((D, D), cst2),
            pl.BlockSpec((D, D), cst2),
            pl.BlockSpec((D, D), cst2),
            pl.BlockSpec((R, D), cst2),
            pl.BlockSpec((NUM_NF, D), cst2),
            pl.BlockSpec((NUM_NF, D), cst2),
            pl.BlockSpec((NUM_NF, D, D), cst3),
            pl.BlockSpec((NUM_NF, D), cst2),
            pl.BlockSpec((NUM_NF, D, D), cst3),
            pl.BlockSpec((NUM_NF, D), cst2),
        ],
        out_specs=[
            pl.BlockSpec((BN, D), row2),
            pl.BlockSpec((R, D), cst2),
        ],
        out_shape=(
            jax.ShapeDtypeStruct((N, D), jnp.float32),
            jax.ShapeDtypeStruct((R, D), jnp.float32),
        ),
    )(na_weights.reshape(1, 3), node_weights.reshape(1, NUM_NF),
      x, node_feat, Wself, Wrel1, Wrel2, rel_embeds,
      nodeW1.reshape(NUM_NF, D), nodeB1, nodeW2, nodeB2, nodeW3, nodeB3)


def _post_body(naw, deg_r, a1_r, a2_r, pre_r, wop1_r, wop2_r, out_r):
    w = jax.nn.softmax(naw[...], axis=-1)
    inv = 1.0 / jnp.maximum(deg_r[...], 1.0)
    a1 = jnp.concatenate([a1_r[0], a1_r[1]], axis=-1) * inv
    a2 = jnp.concatenate([a2_r[0], a2_r[1]], axis=-1) * inv
    acc = pre_r[...]
    acc += w[0, 0] * jnp.dot(a1, wop1_r[...],
                             preferred_element_type=jnp.float32)
    acc += w[0, 1] * jnp.dot(a2, wop2_r[...],
                             preferred_element_type=jnp.float32)
    out_r[...] = jnp.maximum(acc, 0.0)


def _dense_post(na_weights, deg, a1, a2, pre, Wop1, Wop2):
    cst2 = lambda i: (0, 0)
    row2 = lambda i: (i, 0)
    half = lambda i: (0, i, 0)
    return pl.pallas_call(
        _post_body,
        grid=(N // BN,),
        in_specs=[
            pl.BlockSpec((1, 3), cst2),
            pl.BlockSpec((BN, 1), row2),
            pl.BlockSpec((NC, BN, DH), half),
            pl.BlockSpec((NC, BN, DH), half),
            pl.BlockSpec((BN, D), row2),
            pl.BlockSpec((D, D), cst2),
            pl.BlockSpec((D, D), cst2),
        ],
        out_specs=pl.BlockSpec((BN, D), row2),
        out_shape=jax.ShapeDtypeStruct((N, D), jnp.float32),
    )(na_weights.reshape(1, 3), deg, a1, a2, pre, Wop1, Wop2)


def kernel(x, edge_index, edge_type, rel_embeds, na_weights, node_weights,
           node_feat, Wop1, Wop2, Wself, Wrel1, Wrel2,
           nodeW1, nodeB1, nodeW2, nodeB2, nodeW3, nodeB3):
    src3 = edge_index[0].reshape(NCHUNKS, CR, EW)
    dst3 = edge_index[1].reshape(NCHUNKS, CR, EW)
    typ3 = edge_type.reshape(NCHUNKS, CR, EW)
    eil = jnp.concatenate(
        [jnp.stack([src3, dst3, typ3], axis=1),
         jnp.zeros((EPAD, 3, CR, EW), jnp.int32)])
    eil = eil.reshape(NCHUNKS + EPAD, 3 * CR, EW)
    x2 = x.reshape(N, NC, DH).transpose(1, 0, 2)
    rel2 = rel_embeds.reshape(R, NC, DH).transpose(1, 0, 2)

    a1, a2, dego = _sc_edge_pass(x2, rel2, eil)
    pre, relmix = _dense_pre(na_weights, node_weights, x, node_feat, Wself,
                             Wrel1, Wrel2, rel_embeds, nodeW1, nodeB1,
                             nodeW2, nodeB2, nodeW3, nodeB3)
    deg = dego.reshape(DEGT * 1024)[:N].reshape(N, 1)
    xout = _dense_post(na_weights, deg, a1, a2, pre, Wop1, Wop2)
    return xout, relmix
